# Initial kernel scaffold; baseline (speedup 1.0000x reference)
import jax
import jax.numpy as jnp
from jax.experimental import pallas as pl


def kernel(p000, p001, p002, p003, p004, p005, p006, p007, p008, p009, p010, p011, p012, p013, p014, p015, p016, p017, p018, p019, p020, p021, p022, p023, p024, p025, p026, p027, p028, p029, p030, p031, p032, p033, p034, p035, p036, p037, p038, p039, p040, p041, p042, p043, p044, p045, p046, p047, p048, p049, p050, p051, p052, p053, p054, p055, p056, p057, p058, p059, p060, p061, p062, p063, p064, p065, p066, p067, p068, p069, p070, p071, p072, p073, p074, p075, p076, p077, p078, p079, p080, p081, p082, p083, p084, p085, p086, p087, p088, p089, p090, p091, p092, p093, p094, p095, p096, p097, p098, p099, p100, p101, p102, p103, p104, p105, p106, p107, p108, p109, p110, p111, p112, p113, p114, p115, p116, p117, p118, p119, p120, p121, p122, p123, p124, p125, p126, p127, p128, p129, p130, p131, p132, p133, p134, p135, p136, x):
    raise NotImplementedError("write your pallas kernel here")


# speedup vs baseline: 1.4878x; 1.4878x over previous
import functools
import numpy as np
import jax
import jax.numpy as jnp
from jax.experimental import pallas as pl
from jax.experimental.pallas import tpu as pltpu

_BN_SCALE = float(1.0 / np.sqrt(1.0 + 1e-5))


def _ru(x, m):
    return (x + m - 1) // m * m


_CPARAMS = lambda sem: pltpu.CompilerParams(
    dimension_semantics=sem, vmem_limit_bytes=48 * 1024 * 1024)


def _kchunks(K):
    if K <= 2048:
        return [(0, K)]
    nk = -(-K // 2048)
    tk = _ru(-(-K // nk), 128)
    return [(s, min(s + tk, K)) for s in range(0, K, tk)]


def _gemm_body(*refs, n_pairs, relu, has_res):
    bias_ref = refs[2 * n_pairs]
    res_ref = refs[2 * n_pairs + 1] if has_res else None
    o_ref = refs[-1]
    acc = None
    for p in range(n_pairs):
        a, b = refs[2 * p], refs[2 * p + 1]
        for s, e in _kchunks(a.shape[1]):
            d = jnp.dot(a[:, s:e], b[s:e, :],
                        preferred_element_type=jnp.float32)
            acc = d if acc is None else acc + d
    acc += bias_ref[...]
    if has_res:
        acc += res_ref[...].astype(jnp.float32)
    if relu:
        acc = jnp.maximum(acc, 0.0)
    o_ref[...] = acc.astype(o_ref.dtype)


def _gemm(a_list, b_list, bias, relu=False, residual=None,
          out_dtype=jnp.bfloat16):
    M = a_list[0].shape[0]
    N = b_list[0].shape[1]
    tm = min(256, _ru(M, 8))
    tn = min(256, _ru(N, 128))
    Mp, Np = _ru(M, tm), _ru(N, tn)

    inputs, in_specs = [], []
    for a, b in zip(a_list, b_list):
        K = a.shape[1]
        a_p = a if Mp == M else jnp.pad(a, ((0, Mp - M), (0, 0)))
        b_p = b if Np == N else jnp.pad(b, ((0, 0), (0, Np - N)))
        inputs += [a_p, b_p]
        in_specs += [pl.BlockSpec((tm, K), lambda i, j: (i, 0)),
                     pl.BlockSpec((K, tn), lambda i, j: (0, j))]
    bias_p = jnp.pad(bias.astype(jnp.float32).reshape(1, N), ((0, 0), (0, Np - N)))
    inputs.append(bias_p)
    in_specs.append(pl.BlockSpec((1, tn), lambda i, j: (0, j)))
    has_res = residual is not None
    if has_res:
        r = residual.astype(jnp.bfloat16)
        inputs.append(jnp.pad(r, ((0, Mp - M), (0, Np - N))))
        in_specs.append(pl.BlockSpec((tm, tn), lambda i, j: (i, j)))

    out = pl.pallas_call(
        functools.partial(_gemm_body, n_pairs=len(a_list), relu=relu,
                          has_res=has_res),
        out_shape=jax.ShapeDtypeStruct((Mp, Np), out_dtype),
        grid=(Mp // tm, Np // tn),
        in_specs=in_specs,
        out_specs=pl.BlockSpec((tm, tn), lambda i, j: (i, j)),
        compiler_params=_CPARAMS(("parallel", "parallel")),
    )(*inputs)
    return out[:M, :N]


def _c3_body(x_ref, w_ref, b_ref, *rest, bh, Wo, C, relu, has_res, n_store):
    if has_res:
        res_ref, o_ref = rest
    else:
        (o_ref,) = rest
    h0 = pl.program_id(1) * bh
    parts = []
    for i in range(3):
        rows = x_ref[0, pl.ds(h0 + i, bh), :, :]
        for j in range(3):
            parts.append(rows[:, j:j + Wo, :].reshape(bh * Wo, C))
    a = jnp.concatenate(parts, axis=-1)
    acc = None
    for s, e in _kchunks(9 * C):
        d = jnp.dot(a[:, s:e], w_ref[s:e, :],
                    preferred_element_type=jnp.float32)
        acc = d if acc is None else acc + d
    acc += b_ref[...]
    if n_store:
        acc = acc[:, :n_store]
    if has_res:
        acc += res_ref[...].reshape(acc.shape).astype(jnp.float32)
    if relu:
        acc = jnp.maximum(acc, 0.0)
    o_ref[...] = acc.astype(o_ref.dtype).reshape(o_ref.shape)


def _pick_bh(H, W):
    for bh in range(1, H + 1):
        if H % bh == 0 and bh * W >= 256:
            return bh
    return H


def _conv3x3_s1(x, wmat, bias, relu=False, residual=None, n_store=None,
                out_dtype=jnp.bfloat16):
    N, H, W, C = x.shape
    Cout = wmat.shape[1]
    Np = _ru(Cout, 128)
    xp = jnp.pad(x, ((0, 0), (1, 1), (1, 1), (0, 0)))
    Hp, Wp = H + 2, W + 2
    bh = _pick_bh(H, W)
    tn = min(256, Np)
    wp = wmat if Np == Cout else jnp.pad(wmat, ((0, 0), (0, Np - Cout)))
    bias_p = jnp.pad(bias.astype(jnp.float32).reshape(1, Cout),
                     ((0, 0), (0, Np - Cout)))

    n_out = n_store if n_store else tn
    grid = (N, H // bh, Np // tn)
    in_specs = [
        pl.BlockSpec((1, Hp, Wp, C), lambda n, h, t: (n, 0, 0, 0)),
        pl.BlockSpec((9 * C, tn), lambda n, h, t: (0, t)),
        pl.BlockSpec((1, tn), lambda n, h, t: (0, t)),
    ]
    inputs = [xp, wp, bias_p]
    has_res = residual is not None
    if has_res:
        inputs.append(residual.astype(jnp.bfloat16))
        if n_store:
            in_specs.append(pl.BlockSpec((1, bh, W, n_store),
                                         lambda n, h, t: (n, h, 0, 0)))
        else:
            in_specs.append(pl.BlockSpec((1, bh, W, tn),
                                         lambda n, h, t: (n, h, 0, t)))
    out = pl.pallas_call(
        functools.partial(_c3_body, bh=bh, Wo=W, C=C, relu=relu,
                          has_res=has_res, n_store=n_store),
        out_shape=jax.ShapeDtypeStruct((N, H, W, n_out if n_store else Np),
                                       out_dtype),
        grid=grid,
        in_specs=in_specs,
        out_specs=pl.BlockSpec((1, bh, W, n_out),
                               lambda n, h, t: (n, h, 0, 0 if n_store else t)),
        compiler_params=_CPARAMS(("parallel", "parallel", "parallel")),
    )(*inputs)
    if n_store:
        return out[..., :n_store]
    return out[..., :Cout]


def _pool_body(ee_ref, eo_ref, oe_ref, oo_ref, o_ref, Ho, Wo):
    ee, eo, oe, oo = ee_ref[0], eo_ref[0], oe_ref[0], oo_ref[0]
    m = ee[0:Ho, 0:Wo]
    m = jnp.maximum(m, ee[0:Ho, 1:Wo + 1])
    m = jnp.maximum(m, ee[1:Ho + 1, 0:Wo])
    m = jnp.maximum(m, ee[1:Ho + 1, 1:Wo + 1])
    m = jnp.maximum(m, eo[0:Ho, 0:Wo])
    m = jnp.maximum(m, eo[1:Ho + 1, 0:Wo])
    m = jnp.maximum(m, oe[0:Ho, 0:Wo])
    m = jnp.maximum(m, oe[0:Ho, 1:Wo + 1])
    m = jnp.maximum(m, oo[0:Ho, 0:Wo])
    o_ref[0] = m


def _maxpool(x):
    N, H, W, C = x.shape
    xp = jnp.pad(x, ((0, 0), (1, 1), (1, 1), (0, 0)),
                 constant_values=-jnp.inf)
    Ho, Wo = (H + 2 - 3) // 2 + 1, (W + 2 - 3) // 2 + 1
    planes = [xp[:, i::2, j::2, :] for i in range(2) for j in range(2)]
    Hh, Wh = planes[0].shape[1], planes[0].shape[2]
    specs = [pl.BlockSpec((1, Hh, Wh, C), lambda n: (n, 0, 0, 0))] * 4
    return pl.pallas_call(
        functools.partial(_pool_body, Ho=Ho, Wo=Wo),
        out_shape=jax.ShapeDtypeStruct((N, Ho, Wo, C), x.dtype),
        grid=(N,),
        in_specs=specs,
        out_specs=pl.BlockSpec((1, Ho, Wo, C), lambda n: (n, 0, 0, 0)),
        compiler_params=_CPARAMS(("parallel",)),
    )(*planes)


def _wmat2d(w, bn):
    Cout, Cin, kh, kw = w.shape
    m = jnp.transpose(w, (2, 3, 1, 0)).reshape(kh * kw * Cin, Cout)
    if bn:
        m = m * _BN_SCALE
    return m.astype(jnp.bfloat16)


def _bias_of(b, Cout, bn):
    bias = b if b is not None else jnp.zeros((Cout,), jnp.float32)
    bias = bias.astype(jnp.float32)
    return bias * _BN_SCALE if bn else bias


def _conv(x, w, b, stride=(1, 1), padding=((0, 0), (0, 0)), bn=False,
          relu=False, residual=None):
    N, H, W, Cin = x.shape
    Cout, _, kh, kw = w.shape
    sh, sw = stride
    (pt, pb), (plf, prt) = padding
    bias = _bias_of(b, Cout, bn)
    wmat = _wmat2d(w, bn)

    if kh == 3 and kw == 3 and sh == 1 and pt == 1:
        ns = Cout if (Cout < 128 and Cout % 8 == 0) else None
        return _conv3x3_s1(x, wmat, bias, relu=relu, residual=residual,
                           n_store=ns)

    xp = jnp.pad(x, ((0, 0), (pt, pb), (plf, prt), (0, 0)))
    Hp, Wp = H + pt + pb, W + plf + prt
    Ho, Wo = (Hp - kh) // sh + 1, (Wp - kw) // sw + 1
    M = N * Ho * Wo
    if kh == 1 and kw == 1:
        a = (xp if sh == 1 else xp[:, ::sh, ::sw, :]).reshape(M, Cin)
    else:
        cols = [xp[:, i:i + sh * (Ho - 1) + 1:sh, j:j + sw * (Wo - 1) + 1:sw, :]
                for i in range(kh) for j in range(kw)]
        a = jnp.concatenate(cols, axis=-1).reshape(M, kh * kw * Cin)
    res2 = residual.reshape(M, Cout) if residual is not None else None
    out = _gemm([a], [wmat], bias, relu=relu, residual=res2)
    return out.reshape(N, Ho, Wo, Cout)


def _bottleneck(x, bp, stride):
    if 'down' in bp:
        identity = _conv(x, bp['down'][0], None, (stride, stride),
                         bn=True)
    else:
        identity = x
    y = _conv(x, bp['conv1'][0], None, bn=True, relu=True)
    y = _conv(y, bp['conv2'][0], None, (stride, stride), ((1, 1), (1, 1)),
              bn=True, relu=True)
    return _conv(y, bp['conv3'][0], None, bn=True, relu=True,
                 residual=identity)


def _embed3(w):
    Cout, Cin, kh, kw = w.shape
    out = jnp.zeros((Cout, Cin, 3, 3), w.dtype)
    return out.at[:, :, 3 - kh:, 3 - kw:].set(w)


def _interleave4(a, b, c, d):
    N, H, W, C = a.shape
    y = jnp.stack([jnp.stack([a, c], axis=3), jnp.stack([b, d], axis=3)],
                  axis=2)
    return y.reshape(N, 2 * H, 2 * W, C)


def _up_projection(x, p):
    planes = p['conv'][0].shape[0]
    ws, bs = [], []
    for group in ('unpool_main', 'unpool_res'):
        for key in ('A', 'B', 'C', 'D'):
            w, bb = p[group][key]
            ws.append(_embed3(w))
            bs.append(bb)
    w_f = jnp.concatenate(ws, axis=0)
    b_f = jnp.concatenate(bs, axis=0)
    Cin = x.shape[-1]
    wmat = jnp.transpose(w_f, (2, 3, 1, 0)).reshape(9 * Cin, 8 * planes)
    wmat = (wmat * _BN_SCALE).astype(jnp.bfloat16)
    bias = b_f.astype(jnp.float32) * _BN_SCALE

    y = _conv3x3_s1(x, wmat, bias)
    parts = [y[..., i * planes:(i + 1) * planes] for i in range(8)]
    main = _interleave4(*[jnp.maximum(t, 0) for t in parts[:4]])
    res = _interleave4(*parts[4:])
    return _conv(main, p['conv'][0], p['conv'][1], (1, 1), ((1, 1), (1, 1)),
                 bn=True, relu=True, residual=res)


def _con_conv(x_dec, x_skip, p):
    w, b = p
    N, H, W, Cd = x_dec.shape
    Cs = x_skip.shape[-1]
    Cout = w.shape[0]
    wmat = _wmat2d(w, bn=False)
    out = _gemm([x_skip.reshape(N * H * W, Cs), x_dec.reshape(N * H * W, Cd)],
                [wmat[:Cs], wmat[Cs:]], b.astype(jnp.float32))
    return out.reshape(N, H, W, Cout)


def _forward(P, x_nchw):
    x = jnp.transpose(x_nchw, (0, 2, 3, 1)).astype(jnp.bfloat16)
    x = _conv(x, P['conv1'][0], None, (2, 2), ((3, 3), (3, 3)),
              bn=True, relu=True)
    x_to4 = x
    x = _maxpool(x)
    x_to3 = x
    for bi, bp in enumerate(P['layer1']):
        x_to3 = _bottleneck(x_to3, bp, 1)
    x_to2 = x_to3
    for bi, bp in enumerate(P['layer2']):
        x_to2 = _bottleneck(x_to2, bp, 2 if bi == 0 else 1)
    x_to1 = x_to2
    for bi, bp in enumerate(P['layer3']):
        x_to1 = _bottleneck(x_to1, bp, 2 if bi == 0 else 1)
    x = x_to1
    for bi, bp in enumerate(P['layer4']):
        x = _bottleneck(x, bp, 2 if bi == 0 else 1)
    x = _conv(x, P['conv2'][0], P['conv2'][1], bn=True)
    x = _up_projection(x, P['up_proj1'])
    x = _con_conv(x, x_to1, P['con_conv1'])
    x = _up_projection(x, P['up_proj2'])
    x = _con_conv(x, x_to2, P['con_conv2'])
    x = _up_projection(x, P['up_proj3'])
    x = _con_conv(x, x_to3, P['con_conv3'])
    x = _up_projection(x, P['up_proj4'])
    x = _con_conv(x, x_to4, P['con_conv4'])
    wmat = _wmat2d(P['conv3'][0], bn=False)
    out = _conv3x3_s1(x, wmat, P['conv3'][1].astype(jnp.float32),
                      relu=True, n_store=8, out_dtype=jnp.bfloat16)
    out = out[..., :1]
    return jnp.transpose(out, (0, 3, 1, 2)).astype(jnp.float32)


_BLOCKS = (('layer1', 3), ('layer2', 4), ('layer3', 6), ('layer4', 3))


def _unpack(leaves):
    it = iter(leaves)
    nxt = lambda: next(it)
    P = {'conv1': (nxt(), None)}
    for lname, nb in _BLOCKS:
        blocks = []
        for bi in range(nb):
            bp = {'conv1': (nxt(), None), 'conv2': (nxt(), None),
                  'conv3': (nxt(), None)}
            if bi == 0:
                bp['down'] = (nxt(), None)
            blocks.append(bp)
        P[lname] = blocks
    P['conv2'] = (nxt(), nxt())
    for up in ('up_proj1', 'up_proj2', 'up_proj3', 'up_proj4'):
        p = {'unpool_main': {}, 'unpool_res': {}}
        for g in ('unpool_main', 'unpool_res'):
            for k in ('A', 'B', 'C', 'D'):
                p[g][k] = (nxt(), nxt())
        p['conv'] = (nxt(), nxt())
        P[up] = p
    P['conv3'] = (nxt(), nxt())
    for c in ('con_conv1', 'con_conv2', 'con_conv3', 'con_conv4'):
        P[c] = (nxt(), nxt())
    return P


def kernel(*args):
    leaves = list(args[:137])
    x = args[137]
    return _forward(_unpack(leaves), x)


# speedup vs baseline: 1.5353x; 1.0319x over previous
import functools
import numpy as np
import jax
import jax.numpy as jnp
from jax.experimental import pallas as pl
from jax.experimental.pallas import tpu as pltpu

_BN_SCALE = float(1.0 / np.sqrt(1.0 + 1e-5))


def _ru(x, m):
    return (x + m - 1) // m * m


_CPARAMS = lambda sem: pltpu.CompilerParams(
    dimension_semantics=sem, vmem_limit_bytes=48 * 1024 * 1024)


def _kchunks(K):
    if K <= 2048:
        return [(0, K)]
    nk = -(-K // 2048)
    tk = _ru(-(-K // nk), 128)
    return [(s, min(s + tk, K)) for s in range(0, K, tk)]


def _gemm_body(*refs, n_pairs, relu, has_res):
    bias_ref = refs[2 * n_pairs]
    res_ref = refs[2 * n_pairs + 1] if has_res else None
    o_ref = refs[-1]
    acc = None
    for p in range(n_pairs):
        a, b = refs[2 * p], refs[2 * p + 1]
        for s, e in _kchunks(a.shape[1]):
            d = jnp.dot(a[:, s:e], b[s:e, :],
                        preferred_element_type=jnp.float32)
            acc = d if acc is None else acc + d
    acc += bias_ref[...]
    if has_res:
        acc += res_ref[...].astype(jnp.float32)
    if relu:
        acc = jnp.maximum(acc, 0.0)
    o_ref[...] = acc.astype(o_ref.dtype)


def _gemm(a_list, b_list, bias, relu=False, residual=None,
          out_dtype=jnp.bfloat16):
    M = a_list[0].shape[0]
    N = b_list[0].shape[1]
    tm = min(256, _ru(M, 8))
    tn = min(256, _ru(N, 128))
    Mp, Np = _ru(M, tm), _ru(N, tn)

    inputs, in_specs = [], []
    for a, b in zip(a_list, b_list):
        K = a.shape[1]
        a_p = a if Mp == M else jnp.pad(a, ((0, Mp - M), (0, 0)))
        b_p = b if Np == N else jnp.pad(b, ((0, 0), (0, Np - N)))
        inputs += [a_p, b_p]
        in_specs += [pl.BlockSpec((tm, K), lambda i, j: (i, 0)),
                     pl.BlockSpec((K, tn), lambda i, j: (0, j))]
    bias_p = jnp.pad(bias.astype(jnp.float32).reshape(1, N), ((0, 0), (0, Np - N)))
    inputs.append(bias_p)
    in_specs.append(pl.BlockSpec((1, tn), lambda i, j: (0, j)))
    has_res = residual is not None
    if has_res:
        r = residual.astype(jnp.bfloat16)
        inputs.append(jnp.pad(r, ((0, Mp - M), (0, Np - N))))
        in_specs.append(pl.BlockSpec((tm, tn), lambda i, j: (i, j)))

    out = pl.pallas_call(
        functools.partial(_gemm_body, n_pairs=len(a_list), relu=relu,
                          has_res=has_res),
        out_shape=jax.ShapeDtypeStruct((Mp, Np), out_dtype),
        grid=(Mp // tm, Np // tn),
        in_specs=in_specs,
        out_specs=pl.BlockSpec((tm, tn), lambda i, j: (i, j)),
        compiler_params=_CPARAMS(("parallel", "parallel")),
    )(*inputs)
    return out[:M, :N]


def _c3_body(x_ref, w_ref, b_ref, *rest, bh, Wo, C, relu, has_res, n_store):
    if has_res:
        res_ref, o_ref = rest
    else:
        (o_ref,) = rest
    h0 = pl.program_id(1) * bh
    parts = []
    for i in range(3):
        rows = x_ref[0, pl.ds(h0 + i, bh), :, :]
        for j in range(3):
            parts.append(rows[:, j:j + Wo, :].reshape(bh * Wo, C))
    a = jnp.concatenate(parts, axis=-1)
    acc = None
    for s, e in _kchunks(9 * C):
        d = jnp.dot(a[:, s:e], w_ref[s:e, :],
                    preferred_element_type=jnp.float32)
        acc = d if acc is None else acc + d
    acc += b_ref[...]
    if n_store:
        acc = acc[:, :n_store]
    if has_res:
        acc += res_ref[...].reshape(acc.shape).astype(jnp.float32)
    if relu:
        acc = jnp.maximum(acc, 0.0)
    o_ref[...] = acc.astype(o_ref.dtype).reshape(o_ref.shape)


def _pick_bh(H, W):
    for bh in range(1, H + 1):
        if H % bh == 0 and bh * W >= 256:
            return bh
    return H


def _conv3x3_s1(x, wmat, bias, relu=False, residual=None, n_store=None,
                out_dtype=jnp.bfloat16):
    N, H, W, C = x.shape
    Cout = wmat.shape[1]
    Np = _ru(Cout, 128)
    xp = jnp.pad(x, ((0, 0), (1, 1), (1, 1), (0, 0)))
    Hp, Wp = H + 2, W + 2
    bh = _pick_bh(H, W)
    tn = min(256, Np)
    wp = wmat if Np == Cout else jnp.pad(wmat, ((0, 0), (0, Np - Cout)))
    bias_p = jnp.pad(bias.astype(jnp.float32).reshape(1, Cout),
                     ((0, 0), (0, Np - Cout)))

    n_out = n_store if n_store else tn
    grid = (N, H // bh, Np // tn)
    in_specs = [
        pl.BlockSpec((1, Hp, Wp, C), lambda n, h, t: (n, 0, 0, 0)),
        pl.BlockSpec((9 * C, tn), lambda n, h, t: (0, t)),
        pl.BlockSpec((1, tn), lambda n, h, t: (0, t)),
    ]
    inputs = [xp, wp, bias_p]
    has_res = residual is not None
    if has_res:
        inputs.append(residual.astype(jnp.bfloat16))
        if n_store:
            in_specs.append(pl.BlockSpec((1, bh, W, n_store),
                                         lambda n, h, t: (n, h, 0, 0)))
        else:
            in_specs.append(pl.BlockSpec((1, bh, W, tn),
                                         lambda n, h, t: (n, h, 0, t)))
    out = pl.pallas_call(
        functools.partial(_c3_body, bh=bh, Wo=W, C=C, relu=relu,
                          has_res=has_res, n_store=n_store),
        out_shape=jax.ShapeDtypeStruct((N, H, W, n_out if n_store else Np),
                                       out_dtype),
        grid=grid,
        in_specs=in_specs,
        out_specs=pl.BlockSpec((1, bh, W, n_out),
                               lambda n, h, t: (n, h, 0, 0 if n_store else t)),
        compiler_params=_CPARAMS(("parallel", "parallel", "parallel")),
    )(*inputs)
    if n_store:
        return out[..., :n_store]
    return out[..., :Cout]


def _pool_body(ee_ref, eo_ref, oe_ref, oo_ref, o_ref, Ho, Wo):
    ee, eo, oe, oo = ee_ref[0], eo_ref[0], oe_ref[0], oo_ref[0]
    m = ee[0:Ho, 0:Wo]
    m = jnp.maximum(m, ee[0:Ho, 1:Wo + 1])
    m = jnp.maximum(m, ee[1:Ho + 1, 0:Wo])
    m = jnp.maximum(m, ee[1:Ho + 1, 1:Wo + 1])
    m = jnp.maximum(m, eo[0:Ho, 0:Wo])
    m = jnp.maximum(m, eo[1:Ho + 1, 0:Wo])
    m = jnp.maximum(m, oe[0:Ho, 0:Wo])
    m = jnp.maximum(m, oe[0:Ho, 1:Wo + 1])
    m = jnp.maximum(m, oo[0:Ho, 0:Wo])
    o_ref[0] = m


def _maxpool(x):
    N, H, W, C = x.shape
    xp = jnp.pad(x, ((0, 0), (1, 1), (1, 1), (0, 0)),
                 constant_values=-jnp.inf)
    Ho, Wo = (H + 2 - 3) // 2 + 1, (W + 2 - 3) // 2 + 1
    planes = [xp[:, i::2, j::2, :] for i in range(2) for j in range(2)]
    Hh, Wh = planes[0].shape[1], planes[0].shape[2]
    specs = [pl.BlockSpec((1, Hh, Wh, C), lambda n: (n, 0, 0, 0))] * 4
    return pl.pallas_call(
        functools.partial(_pool_body, Ho=Ho, Wo=Wo),
        out_shape=jax.ShapeDtypeStruct((N, Ho, Wo, C), x.dtype),
        grid=(N,),
        in_specs=specs,
        out_specs=pl.BlockSpec((1, Ho, Wo, C), lambda n: (n, 0, 0, 0)),
        compiler_params=_CPARAMS(("parallel",)),
    )(*planes)


def _wmat2d(w, bn):
    Cout, Cin, kh, kw = w.shape
    m = jnp.transpose(w, (2, 3, 1, 0)).reshape(kh * kw * Cin, Cout)
    if bn:
        m = m * _BN_SCALE
    return m.astype(jnp.bfloat16)


def _bias_of(b, Cout, bn):
    bias = b if b is not None else jnp.zeros((Cout,), jnp.float32)
    bias = bias.astype(jnp.float32)
    return bias * _BN_SCALE if bn else bias


def _conv(x, w, b, stride=(1, 1), padding=((0, 0), (0, 0)), bn=False,
          relu=False, residual=None):
    N, H, W, Cin = x.shape
    Cout, _, kh, kw = w.shape
    sh, sw = stride
    (pt, pb), (plf, prt) = padding
    bias = _bias_of(b, Cout, bn)
    wmat = _wmat2d(w, bn)

    if kh == 3 and kw == 3 and sh == 1 and pt == 1:
        ns = Cout if (Cout < 128 and Cout % 8 == 0) else None
        return _conv3x3_s1(x, wmat, bias, relu=relu, residual=residual,
                           n_store=ns)

    xp = jnp.pad(x, ((0, 0), (pt, pb), (plf, prt), (0, 0)))
    Hp, Wp = H + pt + pb, W + plf + prt
    Ho, Wo = (Hp - kh) // sh + 1, (Wp - kw) // sw + 1
    M = N * Ho * Wo
    if kh == 1 and kw == 1:
        a = (xp if sh == 1 else xp[:, ::sh, ::sw, :]).reshape(M, Cin)
    else:
        cols = [xp[:, i:i + sh * (Ho - 1) + 1:sh, j:j + sw * (Wo - 1) + 1:sw, :]
                for i in range(kh) for j in range(kw)]
        a = jnp.concatenate(cols, axis=-1).reshape(M, kh * kw * Cin)
    res2 = residual.reshape(M, Cout) if residual is not None else None
    out = _gemm([a], [wmat], bias, relu=relu, residual=res2)
    return out.reshape(N, Ho, Wo, Cout)


def _bottleneck(x, bp, stride):
    if 'down' in bp:
        identity = _conv(x, bp['down'][0], None, (stride, stride),
                         bn=True)
    else:
        identity = x
    y = _conv(x, bp['conv1'][0], None, bn=True, relu=True)
    y = _conv(y, bp['conv2'][0], None, (stride, stride), ((1, 1), (1, 1)),
              bn=True, relu=True)
    return _conv(y, bp['conv3'][0], None, bn=True, relu=True,
                 residual=identity)


def _embed3(w):
    Cout, Cin, kh, kw = w.shape
    out = jnp.zeros((Cout, Cin, 3, 3), w.dtype)
    return out.at[:, :, 3 - kh:, 3 - kw:].set(w)


def _upproj_body(x_ref, *refs, bh, Wo, C):
    ws, bs = refs[0:8], refs[8:16]
    main_ref, res_ref = refs[16], refs[17]
    h0 = pl.program_id(2) * bh
    parts = []
    for i in range(3):
        rows = x_ref[0, pl.ds(h0 + i, bh), :, :]
        for j in range(3):
            parts.append(rows[:, j:j + Wo, :].reshape(bh * Wo, C))
    a = jnp.concatenate(parts, axis=-1)
    ys = []
    for g in range(8):
        acc = None
        for s, e in _kchunks(9 * C):
            d = jnp.dot(a[:, s:e], ws[g][s:e, :],
                        preferred_element_type=jnp.float32)
            acc = d if acc is None else acc + d
        ys.append(acc + bs[g][...])
    tc = ys[0].shape[-1]

    def weave(ts, relu):
        if relu:
            ts = [jnp.maximum(t, 0.0) for t in ts]
        A, B, Cc, D = [t.astype(main_ref.dtype).reshape(bh, Wo, tc)
                       for t in ts]
        r0 = jnp.concatenate([A[:, :, None, :], Cc[:, :, None, :]],
                             axis=2).reshape(bh, 2 * Wo, tc)
        r1 = jnp.concatenate([B[:, :, None, :], D[:, :, None, :]],
                             axis=2).reshape(bh, 2 * Wo, tc)
        return jnp.concatenate([r0[:, None], r1[:, None]],
                               axis=1).reshape(1, 2 * bh, 2 * Wo, tc)

    main_ref[...] = weave(ys[:4], True)
    res_ref[...] = weave(ys[4:], False)


def _up_projection(x, p):
    planes = p['conv'][0].shape[0]
    N, H, W, C = x.shape
    xp = jnp.pad(x, ((0, 0), (1, 1), (1, 1), (0, 0)))
    ws, bs = [], []
    for group in ('unpool_main', 'unpool_res'):
        for key in ('A', 'B', 'C', 'D'):
            w, bb = p[group][key]
            wm = jnp.transpose(_embed3(w), (2, 3, 1, 0)).reshape(9 * C, planes)
            ws.append((wm * _BN_SCALE).astype(jnp.bfloat16))
            bs.append(bb.astype(jnp.float32).reshape(1, planes) * _BN_SCALE)
    tc = min(128, planes)
    bh = _pick_bh(H, W)
    grid = (planes // tc, N, H // bh)
    in_specs = [pl.BlockSpec((1, H + 2, W + 2, C), lambda t, n, h: (n, 0, 0, 0))]
    in_specs += [pl.BlockSpec((9 * C, tc), lambda t, n, h: (0, t))] * 8
    in_specs += [pl.BlockSpec((1, tc), lambda t, n, h: (0, t))] * 8
    out_sds = jax.ShapeDtypeStruct((N, 2 * H, 2 * W, planes), jnp.bfloat16)
    out_spec = pl.BlockSpec((1, 2 * bh, 2 * W, tc), lambda t, n, h: (n, h, 0, t))
    main, res = pl.pallas_call(
        functools.partial(_upproj_body, bh=bh, Wo=W, C=C),
        out_shape=[out_sds, out_sds],
        grid=grid,
        in_specs=in_specs,
        out_specs=[out_spec, out_spec],
        compiler_params=_CPARAMS(("parallel", "parallel", "parallel")),
    )(xp, *ws, *bs)
    return _conv(main, p['conv'][0], p['conv'][1], (1, 1), ((1, 1), (1, 1)),
                 bn=True, relu=True, residual=res)


def _con_conv(x_dec, x_skip, p):
    w, b = p
    N, H, W, Cd = x_dec.shape
    Cs = x_skip.shape[-1]
    Cout = w.shape[0]
    wmat = _wmat2d(w, bn=False)
    out = _gemm([x_skip.reshape(N * H * W, Cs), x_dec.reshape(N * H * W, Cd)],
                [wmat[:Cs], wmat[Cs:]], b.astype(jnp.float32))
    return out.reshape(N, H, W, Cout)


def _forward(P, x_nchw):
    x = jnp.transpose(x_nchw, (0, 2, 3, 1)).astype(jnp.bfloat16)
    x = _conv(x, P['conv1'][0], None, (2, 2), ((3, 3), (3, 3)),
              bn=True, relu=True)
    x_to4 = x
    x = _maxpool(x)
    x_to3 = x
    for bi, bp in enumerate(P['layer1']):
        x_to3 = _bottleneck(x_to3, bp, 1)
    x_to2 = x_to3
    for bi, bp in enumerate(P['layer2']):
        x_to2 = _bottleneck(x_to2, bp, 2 if bi == 0 else 1)
    x_to1 = x_to2
    for bi, bp in enumerate(P['layer3']):
        x_to1 = _bottleneck(x_to1, bp, 2 if bi == 0 else 1)
    x = x_to1
    for bi, bp in enumerate(P['layer4']):
        x = _bottleneck(x, bp, 2 if bi == 0 else 1)
    x = _conv(x, P['conv2'][0], P['conv2'][1], bn=True)
    x = _up_projection(x, P['up_proj1'])
    x = _con_conv(x, x_to1, P['con_conv1'])
    x = _up_projection(x, P['up_proj2'])
    x = _con_conv(x, x_to2, P['con_conv2'])
    x = _up_projection(x, P['up_proj3'])
    x = _con_conv(x, x_to3, P['con_conv3'])
    x = _up_projection(x, P['up_proj4'])
    x = _con_conv(x, x_to4, P['con_conv4'])
    wmat = _wmat2d(P['conv3'][0], bn=False)
    out = _conv3x3_s1(x, wmat, P['conv3'][1].astype(jnp.float32),
                      relu=True, n_store=8, out_dtype=jnp.bfloat16)
    out = out[..., :1]
    return jnp.transpose(out, (0, 3, 1, 2)).astype(jnp.float32)


_BLOCKS = (('layer1', 3), ('layer2', 4), ('layer3', 6), ('layer4', 3))


def _unpack(leaves):
    it = iter(leaves)
    nxt = lambda: next(it)
    P = {'conv1': (nxt(), None)}
    for lname, nb in _BLOCKS:
        blocks = []
        for bi in range(nb):
            bp = {'conv1': (nxt(), None), 'conv2': (nxt(), None),
                  'conv3': (nxt(), None)}
            if bi == 0:
                bp['down'] = (nxt(), None)
            blocks.append(bp)
        P[lname] = blocks
    P['conv2'] = (nxt(), nxt())
    for up in ('up_proj1', 'up_proj2', 'up_proj3', 'up_proj4'):
        p = {'unpool_main': {}, 'unpool_res': {}}
        for g in ('unpool_main', 'unpool_res'):
            for k in ('A', 'B', 'C', 'D'):
                p[g][k] = (nxt(), nxt())
        p['conv'] = (nxt(), nxt())
        P[up] = p
    P['conv3'] = (nxt(), nxt())
    for c in ('con_conv1', 'con_conv2', 'con_conv3', 'con_conv4'):
        P[c] = (nxt(), nxt())
    return P


def kernel(*args):
    leaves = list(args[:137])
    x = args[137]
    return _forward(_unpack(leaves), x)


# speedup vs baseline: 2.5345x; 1.6508x over previous
import functools
import numpy as np
import jax
import jax.numpy as jnp
from jax.experimental import pallas as pl
from jax.experimental.pallas import tpu as pltpu

_BN_SCALE = float(1.0 / np.sqrt(1.0 + 1e-5))


def _ru(x, m):
    return (x + m - 1) // m * m


_CPARAMS = lambda sem: pltpu.CompilerParams(
    dimension_semantics=sem, vmem_limit_bytes=48 * 1024 * 1024)


def _kchunks(K):
    if K <= 2048:
        return [(0, K)]
    nk = -(-K // 2048)
    tk = _ru(-(-K // nk), 128)
    return [(s, min(s + tk, K)) for s in range(0, K, tk)]


def _gemm_body(*refs, n_pairs, relu, has_res):
    bias_ref = refs[2 * n_pairs]
    res_ref = refs[2 * n_pairs + 1] if has_res else None
    o_ref = refs[-1]
    acc = None
    for p in range(n_pairs):
        a, b = refs[2 * p], refs[2 * p + 1]
        for s, e in _kchunks(a.shape[1]):
            d = jnp.dot(a[:, s:e], b[s:e, :],
                        preferred_element_type=jnp.float32)
            acc = d if acc is None else acc + d
    acc += bias_ref[...]
    if has_res:
        acc += res_ref[...].astype(jnp.float32)
    if relu:
        acc = jnp.maximum(acc, 0.0)
    o_ref[...] = acc.astype(o_ref.dtype)


def _gemm(a_list, b_list, bias, relu=False, residual=None,
          out_dtype=jnp.bfloat16):
    M = a_list[0].shape[0]
    N = b_list[0].shape[1]
    tm = min(256, _ru(M, 8))
    tn = min(256, _ru(N, 128))
    Mp, Np = _ru(M, tm), _ru(N, tn)

    inputs, in_specs = [], []
    for a, b in zip(a_list, b_list):
        K = a.shape[1]
        a_p = a if Mp == M else jnp.pad(a, ((0, Mp - M), (0, 0)))
        b_p = b if Np == N else jnp.pad(b, ((0, 0), (0, Np - N)))
        inputs += [a_p, b_p]
        in_specs += [pl.BlockSpec((tm, K), lambda i, j: (i, 0)),
                     pl.BlockSpec((K, tn), lambda i, j: (0, j))]
    bias_p = jnp.pad(bias.astype(jnp.float32).reshape(1, N), ((0, 0), (0, Np - N)))
    inputs.append(bias_p)
    in_specs.append(pl.BlockSpec((1, tn), lambda i, j: (0, j)))
    has_res = residual is not None
    if has_res:
        r = residual.astype(jnp.bfloat16)
        inputs.append(jnp.pad(r, ((0, Mp - M), (0, Np - N))))
        in_specs.append(pl.BlockSpec((tm, tn), lambda i, j: (i, j)))

    out = pl.pallas_call(
        functools.partial(_gemm_body, n_pairs=len(a_list), relu=relu,
                          has_res=has_res),
        out_shape=jax.ShapeDtypeStruct((Mp, Np), out_dtype),
        grid=(Mp // tm, Np // tn),
        in_specs=in_specs,
        out_specs=pl.BlockSpec((tm, tn), lambda i, j: (i, j)),
        compiler_params=_CPARAMS(("parallel", "parallel")),
    )(*inputs)
    return out[:M, :N]


def _tap_tiles(x_ref, h0, bh, Wo, C, kh, kw, stride):
    pieces = []
    if stride == 1:
        for i in range(kh):
            rows = x_ref[0, pl.ds(h0 + i, bh), :, :]
            for j in range(kw):
                pieces.append(rows[:, j:j + Wo, :].reshape(bh * Wo, C))
    else:
        hneed = 2 * bh + 2 * ((kh - 1) // 2) + (2 if kh % 2 == 0 else 0)
        vv = x_ref[0, pl.ds(2 * h0, hneed), :, :]
        Wp = vv.shape[1]
        r = vv.reshape(hneed // 2, 2, Wp // 2, 2, C)
        for i in range(kh):
            for j in range(kw):
                p = r[i // 2:i // 2 + bh, i % 2,
                      j // 2:j // 2 + Wo, j % 2, :]
                pieces.append(p.reshape(bh * Wo, C))
    return pieces


def _c3_body(x_ref, w_ref, b_ref, *rest, bh, Wo, C, kh, kw, stride, relu,
             has_res, n_store):
    if has_res:
        res_ref, o_ref = rest
    else:
        (o_ref,) = rest
    h0 = pl.program_id(1) * bh
    parts = _tap_tiles(x_ref, h0, bh, Wo, C, kh, kw, stride)
    a = parts[0] if len(parts) == 1 else jnp.concatenate(parts, axis=-1)
    acc = None
    for s, e in _kchunks(kh * kw * C):
        d = jnp.dot(a[:, s:e], w_ref[s:e, :],
                    preferred_element_type=jnp.float32)
        acc = d if acc is None else acc + d
    acc += b_ref[...]
    if n_store:
        acc = acc[:, :n_store]
    if has_res:
        acc += res_ref[...].reshape(acc.shape).astype(jnp.float32)
    if relu:
        acc = jnp.maximum(acc, 0.0)
    o_ref[...] = acc.astype(o_ref.dtype).reshape(o_ref.shape)


def _pick_bh(H, W):
    for bh in range(1, H + 1):
        if H % bh == 0 and bh * W >= 256:
            return bh
    return H


def _conv_implicit(x, wmat, bias, kh=3, kw=3, stride=1, pad=1, relu=False,
                   residual=None, n_store=None, out_dtype=jnp.bfloat16):
    N, H, W, C = x.shape
    Cout = wmat.shape[1]
    Np = _ru(Cout, 128)
    xp = (jnp.pad(x, ((0, 0), (pad, pad), (pad, pad), (0, 0)))
          if pad else x)
    Hp, Wp = H + 2 * pad, W + 2 * pad
    Ho, Wo = (Hp - kh) // stride + 1, (Wp - kw) // stride + 1
    bh = _pick_bh(Ho, Wo)
    tn = min(256, Np)
    wp = wmat if Np == Cout else jnp.pad(wmat, ((0, 0), (0, Np - Cout)))
    bias_p = jnp.pad(bias.astype(jnp.float32).reshape(1, Cout),
                     ((0, 0), (0, Np - Cout)))

    n_out = n_store if n_store else tn
    grid = (N, Ho // bh, Np // tn)
    in_specs = [
        pl.BlockSpec((1, Hp, Wp, C), lambda n, h, t: (n, 0, 0, 0)),
        pl.BlockSpec((kh * kw * C, tn), lambda n, h, t: (0, t)),
        pl.BlockSpec((1, tn), lambda n, h, t: (0, t)),
    ]
    inputs = [xp, wp, bias_p]
    has_res = residual is not None
    if has_res:
        inputs.append(residual.astype(jnp.bfloat16))
        if n_store:
            in_specs.append(pl.BlockSpec((1, bh, Wo, n_store),
                                         lambda n, h, t: (n, h, 0, 0)))
        else:
            in_specs.append(pl.BlockSpec((1, bh, Wo, tn),
                                         lambda n, h, t: (n, h, 0, t)))
    out = pl.pallas_call(
        functools.partial(_c3_body, bh=bh, Wo=Wo, C=C, kh=kh, kw=kw,
                          stride=stride, relu=relu,
                          has_res=has_res, n_store=n_store),
        out_shape=jax.ShapeDtypeStruct((N, Ho, Wo, n_out if n_store else Np),
                                       out_dtype),
        grid=grid,
        in_specs=in_specs,
        out_specs=pl.BlockSpec((1, bh, Wo, n_out),
                               lambda n, h, t: (n, h, 0, 0 if n_store else t)),
        compiler_params=_CPARAMS(("parallel", "parallel", "parallel")),
    )(*inputs)
    if n_store:
        return out[..., :n_store]
    return out[..., :Cout]


def _pool_body(x_ref, o_ref, Ho, Wo):
    v = x_ref[0]
    Hp, Wp, C = v.shape
    r = v.reshape(Hp // 2, 2, Wp // 2, 2, C)
    ee, eo, oe, oo = (r[:, 0, :, 0, :], r[:, 0, :, 1, :],
                      r[:, 1, :, 0, :], r[:, 1, :, 1, :])
    m = ee[0:Ho, 0:Wo]
    m = jnp.maximum(m, ee[0:Ho, 1:Wo + 1])
    m = jnp.maximum(m, ee[1:Ho + 1, 0:Wo])
    m = jnp.maximum(m, ee[1:Ho + 1, 1:Wo + 1])
    m = jnp.maximum(m, eo[0:Ho, 0:Wo])
    m = jnp.maximum(m, eo[1:Ho + 1, 0:Wo])
    m = jnp.maximum(m, oe[0:Ho, 0:Wo])
    m = jnp.maximum(m, oe[0:Ho, 1:Wo + 1])
    m = jnp.maximum(m, oo[0:Ho, 0:Wo])
    o_ref[0] = m


def _maxpool(x):
    N, H, W, C = x.shape
    xp = jnp.pad(x, ((0, 0), (1, 1), (1, 1), (0, 0)),
                 constant_values=-jnp.inf)
    Hp, Wp = H + 2, W + 2
    Ho, Wo = (H + 2 - 3) // 2 + 1, (W + 2 - 3) // 2 + 1
    return pl.pallas_call(
        functools.partial(_pool_body, Ho=Ho, Wo=Wo),
        out_shape=jax.ShapeDtypeStruct((N, Ho, Wo, C), x.dtype),
        grid=(N,),
        in_specs=[pl.BlockSpec((1, Hp, Wp, C), lambda n: (n, 0, 0, 0))],
        out_specs=pl.BlockSpec((1, Ho, Wo, C), lambda n: (n, 0, 0, 0)),
        compiler_params=_CPARAMS(("parallel",)),
    )(xp)


def _wmat2d(w, bn):
    Cout, Cin, kh, kw = w.shape
    m = jnp.transpose(w, (2, 3, 1, 0)).reshape(kh * kw * Cin, Cout)
    if bn:
        m = m * _BN_SCALE
    return m.astype(jnp.bfloat16)


def _bias_of(b, Cout, bn):
    bias = b if b is not None else jnp.zeros((Cout,), jnp.float32)
    bias = bias.astype(jnp.float32)
    return bias * _BN_SCALE if bn else bias


def _conv(x, w, b, stride=(1, 1), padding=((0, 0), (0, 0)), bn=False,
          relu=False, residual=None):
    N, H, W, Cin = x.shape
    Cout, _, kh, kw = w.shape
    sh, sw = stride
    (pt, pb), (plf, prt) = padding
    bias = _bias_of(b, Cout, bn)
    wmat = _wmat2d(w, bn)

    if kh == 1 and kw == 1 and sh == 1:
        M = N * H * W
        a = x.reshape(M, Cin)
        res2 = residual.reshape(M, Cout) if residual is not None else None
        out = _gemm([a], [wmat], bias, relu=relu, residual=res2)
        return out.reshape(N, H, W, Cout)
    ns = Cout if (Cout < 128 and Cout % 8 == 0) else None
    return _conv_implicit(x, wmat, bias, kh=kh, kw=kw, stride=sh, pad=pt,
                          relu=relu, residual=residual, n_store=ns)


def _bottleneck(x, bp, stride):
    if 'down' in bp:
        identity = _conv(x, bp['down'][0], None, (stride, stride),
                         bn=True)
    else:
        identity = x
    y = _conv(x, bp['conv1'][0], None, bn=True, relu=True)
    y = _conv(y, bp['conv2'][0], None, (stride, stride), ((1, 1), (1, 1)),
              bn=True, relu=True)
    return _conv(y, bp['conv3'][0], None, bn=True, relu=True,
                 residual=identity)


def _embed3(w):
    Cout, Cin, kh, kw = w.shape
    out = jnp.zeros((Cout, Cin, 3, 3), w.dtype)
    return out.at[:, :, 3 - kh:, 3 - kw:].set(w)


def _upproj_body(x_ref, *refs, bh, Wo, C):
    ws, bs = refs[0:8], refs[8:16]
    main_ref, res_ref = refs[16], refs[17]
    h0 = pl.program_id(2) * bh
    parts = []
    for i in range(3):
        rows = x_ref[0, pl.ds(h0 + i, bh), :, :]
        for j in range(3):
            parts.append(rows[:, j:j + Wo, :].reshape(bh * Wo, C))
    a = jnp.concatenate(parts, axis=-1)
    ys = []
    for g in range(8):
        acc = None
        for s, e in _kchunks(9 * C):
            d = jnp.dot(a[:, s:e], ws[g][s:e, :],
                        preferred_element_type=jnp.float32)
            acc = d if acc is None else acc + d
        ys.append(acc + bs[g][...])
    tc = ys[0].shape[-1]

    def weave(ts, relu):
        if relu:
            ts = [jnp.maximum(t, 0.0) for t in ts]
        A, B, Cc, D = [t.astype(main_ref.dtype).reshape(bh, Wo, tc)
                       for t in ts]
        r0 = jnp.concatenate([A[:, :, None, :], Cc[:, :, None, :]],
                             axis=2).reshape(bh, 2 * Wo, tc)
        r1 = jnp.concatenate([B[:, :, None, :], D[:, :, None, :]],
                             axis=2).reshape(bh, 2 * Wo, tc)
        return jnp.concatenate([r0[:, None], r1[:, None]],
                               axis=1).reshape(1, 2 * bh, 2 * Wo, tc)

    main_ref[...] = weave(ys[:4], True)
    res_ref[...] = weave(ys[4:], False)


def _up_projection(x, p):
    planes = p['conv'][0].shape[0]
    N, H, W, C = x.shape
    xp = jnp.pad(x, ((0, 0), (1, 1), (1, 1), (0, 0)))
    ws, bs = [], []
    for group in ('unpool_main', 'unpool_res'):
        for key in ('A', 'B', 'C', 'D'):
            w, bb = p[group][key]
            wm = jnp.transpose(_embed3(w), (2, 3, 1, 0)).reshape(9 * C, planes)
            ws.append((wm * _BN_SCALE).astype(jnp.bfloat16))
            bs.append(bb.astype(jnp.float32).reshape(1, planes) * _BN_SCALE)
    tc = min(128, planes)
    bh = _pick_bh(H, W)
    grid = (planes // tc, N, H // bh)
    in_specs = [pl.BlockSpec((1, H + 2, W + 2, C), lambda t, n, h: (n, 0, 0, 0))]
    in_specs += [pl.BlockSpec((9 * C, tc), lambda t, n, h: (0, t))] * 8
    in_specs += [pl.BlockSpec((1, tc), lambda t, n, h: (0, t))] * 8
    out_sds = jax.ShapeDtypeStruct((N, 2 * H, 2 * W, planes), jnp.bfloat16)
    out_spec = pl.BlockSpec((1, 2 * bh, 2 * W, tc), lambda t, n, h: (n, h, 0, t))
    main, res = pl.pallas_call(
        functools.partial(_upproj_body, bh=bh, Wo=W, C=C),
        out_shape=[out_sds, out_sds],
        grid=grid,
        in_specs=in_specs,
        out_specs=[out_spec, out_spec],
        compiler_params=_CPARAMS(("parallel", "parallel", "parallel")),
    )(xp, *ws, *bs)
    return _conv(main, p['conv'][0], p['conv'][1], (1, 1), ((1, 1), (1, 1)),
                 bn=True, relu=True, residual=res)


def _con_conv(x_dec, x_skip, p):
    w, b = p
    N, H, W, Cd = x_dec.shape
    Cs = x_skip.shape[-1]
    Cout = w.shape[0]
    wmat = _wmat2d(w, bn=False)
    out = _gemm([x_skip.reshape(N * H * W, Cs), x_dec.reshape(N * H * W, Cd)],
                [wmat[:Cs], wmat[Cs:]], b.astype(jnp.float32))
    return out.reshape(N, H, W, Cout)


def _forward(P, x_nchw):
    x = jnp.transpose(x_nchw, (0, 2, 3, 1)).astype(jnp.bfloat16)
    x = _conv(x, P['conv1'][0], None, (2, 2), ((3, 3), (3, 3)),
              bn=True, relu=True)
    x_to4 = x
    x = _maxpool(x)
    x_to3 = x
    for bi, bp in enumerate(P['layer1']):
        x_to3 = _bottleneck(x_to3, bp, 1)
    x_to2 = x_to3
    for bi, bp in enumerate(P['layer2']):
        x_to2 = _bottleneck(x_to2, bp, 2 if bi == 0 else 1)
    x_to1 = x_to2
    for bi, bp in enumerate(P['layer3']):
        x_to1 = _bottleneck(x_to1, bp, 2 if bi == 0 else 1)
    x = x_to1
    for bi, bp in enumerate(P['layer4']):
        x = _bottleneck(x, bp, 2 if bi == 0 else 1)
    x = _conv(x, P['conv2'][0], P['conv2'][1], bn=True)
    x = _up_projection(x, P['up_proj1'])
    x = _con_conv(x, x_to1, P['con_conv1'])
    x = _up_projection(x, P['up_proj2'])
    x = _con_conv(x, x_to2, P['con_conv2'])
    x = _up_projection(x, P['up_proj3'])
    x = _con_conv(x, x_to3, P['con_conv3'])
    x = _up_projection(x, P['up_proj4'])
    x = _con_conv(x, x_to4, P['con_conv4'])
    wmat = _wmat2d(P['conv3'][0], bn=False)
    out = _conv_implicit(x, wmat, P['conv3'][1].astype(jnp.float32),
                         relu=True, n_store=8, out_dtype=jnp.bfloat16)
    out = out[..., :1]
    return jnp.transpose(out, (0, 3, 1, 2)).astype(jnp.float32)


_BLOCKS = (('layer1', 3), ('layer2', 4), ('layer3', 6), ('layer4', 3))


def _unpack(leaves):
    it = iter(leaves)
    nxt = lambda: next(it)
    P = {'conv1': (nxt(), None)}
    for lname, nb in _BLOCKS:
        blocks = []
        for bi in range(nb):
            bp = {'conv1': (nxt(), None), 'conv2': (nxt(), None),
                  'conv3': (nxt(), None)}
            if bi == 0:
                bp['down'] = (nxt(), None)
            blocks.append(bp)
        P[lname] = blocks
    P['conv2'] = (nxt(), nxt())
    for up in ('up_proj1', 'up_proj2', 'up_proj3', 'up_proj4'):
        p = {'unpool_main': {}, 'unpool_res': {}}
        for g in ('unpool_main', 'unpool_res'):
            for k in ('A', 'B', 'C', 'D'):
                p[g][k] = (nxt(), nxt())
        p['conv'] = (nxt(), nxt())
        P[up] = p
    P['conv3'] = (nxt(), nxt())
    for c in ('con_conv1', 'con_conv2', 'con_conv3', 'con_conv4'):
        P[c] = (nxt(), nxt())
    return P


def kernel(*args):
    leaves = list(args[:137])
    x = args[137]
    return _forward(_unpack(leaves), x)


# speedup vs baseline: 2.8108x; 1.1090x over previous
import functools
import numpy as np
import jax
import jax.numpy as jnp
from jax.experimental import pallas as pl
from jax.experimental.pallas import tpu as pltpu

_BN_SCALE = float(1.0 / np.sqrt(1.0 + 1e-5))


def _ru(x, m):
    return (x + m - 1) // m * m


_CPARAMS = lambda sem: pltpu.CompilerParams(
    dimension_semantics=sem, vmem_limit_bytes=48 * 1024 * 1024)


def _kchunks(K):
    if K <= 2048:
        return [(0, K)]
    nk = -(-K // 2048)
    tk = _ru(-(-K // nk), 128)
    return [(s, min(s + tk, K)) for s in range(0, K, tk)]


def _gemm_body(*refs, n_pairs, relu, has_res):
    bias_ref = refs[2 * n_pairs]
    res_ref = refs[2 * n_pairs + 1] if has_res else None
    o_ref = refs[-1]
    acc = None
    for p in range(n_pairs):
        a, b = refs[2 * p], refs[2 * p + 1]
        for s, e in _kchunks(a.shape[1]):
            d = jnp.dot(a[:, s:e], b[s:e, :],
                        preferred_element_type=jnp.float32)
            acc = d if acc is None else acc + d
    acc += bias_ref[...]
    if has_res:
        acc += res_ref[...].astype(jnp.float32)
    if relu:
        acc = jnp.maximum(acc, 0.0)
    o_ref[...] = acc.astype(o_ref.dtype)


def _gemm(a_list, b_list, bias, relu=False, residual=None,
          out_dtype=jnp.bfloat16):
    M = a_list[0].shape[0]
    N = b_list[0].shape[1]
    tm = min(256, _ru(M, 8))
    tn = min(256, _ru(N, 128))
    Mp, Np = _ru(M, tm), _ru(N, tn)

    inputs, in_specs = [], []
    for a, b in zip(a_list, b_list):
        K = a.shape[1]
        a_p = a if Mp == M else jnp.pad(a, ((0, Mp - M), (0, 0)))
        b_p = b if Np == N else jnp.pad(b, ((0, 0), (0, Np - N)))
        inputs += [a_p, b_p]
        in_specs += [pl.BlockSpec((tm, K), lambda i, j: (i, 0)),
                     pl.BlockSpec((K, tn), lambda i, j: (0, j))]
    bias_p = jnp.pad(bias.astype(jnp.float32).reshape(1, N), ((0, 0), (0, Np - N)))
    inputs.append(bias_p)
    in_specs.append(pl.BlockSpec((1, tn), lambda i, j: (0, j)))
    has_res = residual is not None
    if has_res:
        r = residual.astype(jnp.bfloat16)
        inputs.append(jnp.pad(r, ((0, Mp - M), (0, Np - N))))
        in_specs.append(pl.BlockSpec((tm, tn), lambda i, j: (i, j)))

    out = pl.pallas_call(
        functools.partial(_gemm_body, n_pairs=len(a_list), relu=relu,
                          has_res=has_res),
        out_shape=jax.ShapeDtypeStruct((Mp, Np), out_dtype),
        grid=(Mp // tm, Np // tn),
        in_specs=in_specs,
        out_specs=pl.BlockSpec((tm, tn), lambda i, j: (i, j)),
        compiler_params=_CPARAMS(("parallel", "parallel")),
    )(*inputs)
    return out[:M, :N]


def _tap_tiles(x_ref, h0, bh, Wo, C, kh, kw, stride):
    pieces = []
    if stride == 1:
        for i in range(kh):
            rows = x_ref[0, pl.ds(h0 + i, bh), :, :]
            for j in range(kw):
                pieces.append(rows[:, j:j + Wo, :].reshape(bh * Wo, C))
    else:
        hneed = 2 * bh + 2 * ((kh - 1) // 2) + (2 if kh % 2 == 0 else 0)
        vv = x_ref[0, pl.ds(2 * h0, hneed), :, :]
        Wp = vv.shape[1]
        r = vv.reshape(hneed // 2, 2, Wp // 2, 2, C)
        for i in range(kh):
            for j in range(kw):
                p = r[i // 2:i // 2 + bh, i % 2,
                      j // 2:j // 2 + Wo, j % 2, :]
                pieces.append(p.reshape(bh * Wo, C))
    return pieces


def _c3_body(x_ref, w_ref, b_ref, *rest, bh, Wo, C, kh, kw, stride, relu,
             has_res, n_store):
    if has_res:
        res_ref, o_ref = rest
    else:
        (o_ref,) = rest
    h0 = pl.program_id(1) * bh
    parts = _tap_tiles(x_ref, h0, bh, Wo, C, kh, kw, stride)
    a = parts[0] if len(parts) == 1 else jnp.concatenate(parts, axis=-1)
    acc = None
    for s, e in _kchunks(kh * kw * C):
        d = jnp.dot(a[:, s:e], w_ref[s:e, :],
                    preferred_element_type=jnp.float32)
        acc = d if acc is None else acc + d
    acc += b_ref[...]
    if n_store:
        acc = acc[:, :n_store]
    if has_res:
        acc += res_ref[...].reshape(acc.shape).astype(jnp.float32)
    if relu:
        acc = jnp.maximum(acc, 0.0)
    o_ref[...] = acc.astype(o_ref.dtype).reshape(o_ref.shape)


def _pick_bh(H, W):
    for bh in range(1, H + 1):
        if H % bh == 0 and bh * W >= 256:
            return bh
    return H


def _conv_implicit(x, wmat, bias, kh=3, kw=3, stride=1, pad=1, relu=False,
                   residual=None, n_store=None, out_dtype=jnp.bfloat16):
    N, H, W, C = x.shape
    Cout = wmat.shape[1]
    Np = _ru(Cout, 128)
    xp = (jnp.pad(x, ((0, 0), (pad, pad), (pad, pad), (0, 0)))
          if pad else x)
    Hp, Wp = H + 2 * pad, W + 2 * pad
    Ho, Wo = (Hp - kh) // stride + 1, (Wp - kw) // stride + 1
    bh = _pick_bh(Ho, Wo)
    tn = min(256, Np)
    wp = wmat if Np == Cout else jnp.pad(wmat, ((0, 0), (0, Np - Cout)))
    bias_p = jnp.pad(bias.astype(jnp.float32).reshape(1, Cout),
                     ((0, 0), (0, Np - Cout)))

    n_out = n_store if n_store else tn
    grid = (N, Ho // bh, Np // tn)
    in_specs = [
        pl.BlockSpec((1, Hp, Wp, C), lambda n, h, t: (n, 0, 0, 0)),
        pl.BlockSpec((kh * kw * C, tn), lambda n, h, t: (0, t)),
        pl.BlockSpec((1, tn), lambda n, h, t: (0, t)),
    ]
    inputs = [xp, wp, bias_p]
    has_res = residual is not None
    if has_res:
        inputs.append(residual.astype(jnp.bfloat16))
        if n_store:
            in_specs.append(pl.BlockSpec((1, bh, Wo, n_store),
                                         lambda n, h, t: (n, h, 0, 0)))
        else:
            in_specs.append(pl.BlockSpec((1, bh, Wo, tn),
                                         lambda n, h, t: (n, h, 0, t)))
    out = pl.pallas_call(
        functools.partial(_c3_body, bh=bh, Wo=Wo, C=C, kh=kh, kw=kw,
                          stride=stride, relu=relu,
                          has_res=has_res, n_store=n_store),
        out_shape=jax.ShapeDtypeStruct((N, Ho, Wo, n_out if n_store else Np),
                                       out_dtype),
        grid=grid,
        in_specs=in_specs,
        out_specs=pl.BlockSpec((1, bh, Wo, n_out),
                               lambda n, h, t: (n, h, 0, 0 if n_store else t)),
        compiler_params=_CPARAMS(("parallel", "parallel", "parallel")),
    )(*inputs)
    if n_store:
        return out[..., :n_store]
    return out[..., :Cout]


def _pool_body(x_ref, o_ref, Ho, Wo):
    v = x_ref[0]
    Hp, Wp, C = v.shape
    r = v.reshape(Hp // 2, 2, Wp // 2, 2, C)
    ee, eo, oe, oo = (r[:, 0, :, 0, :], r[:, 0, :, 1, :],
                      r[:, 1, :, 0, :], r[:, 1, :, 1, :])
    m = ee[0:Ho, 0:Wo]
    m = jnp.maximum(m, ee[0:Ho, 1:Wo + 1])
    m = jnp.maximum(m, ee[1:Ho + 1, 0:Wo])
    m = jnp.maximum(m, ee[1:Ho + 1, 1:Wo + 1])
    m = jnp.maximum(m, eo[0:Ho, 0:Wo])
    m = jnp.maximum(m, eo[1:Ho + 1, 0:Wo])
    m = jnp.maximum(m, oe[0:Ho, 0:Wo])
    m = jnp.maximum(m, oe[0:Ho, 1:Wo + 1])
    m = jnp.maximum(m, oo[0:Ho, 0:Wo])
    o_ref[0] = m


def _maxpool(x):
    N, H, W, C = x.shape
    xp = jnp.pad(x, ((0, 0), (1, 1), (1, 1), (0, 0)),
                 constant_values=-jnp.inf)
    Hp, Wp = H + 2, W + 2
    Ho, Wo = (H + 2 - 3) // 2 + 1, (W + 2 - 3) // 2 + 1
    return pl.pallas_call(
        functools.partial(_pool_body, Ho=Ho, Wo=Wo),
        out_shape=jax.ShapeDtypeStruct((N, Ho, Wo, C), x.dtype),
        grid=(N,),
        in_specs=[pl.BlockSpec((1, Hp, Wp, C), lambda n: (n, 0, 0, 0))],
        out_specs=pl.BlockSpec((1, Ho, Wo, C), lambda n: (n, 0, 0, 0)),
        compiler_params=_CPARAMS(("parallel",)),
    )(xp)


def _wmat2d(w, bn):
    Cout, Cin, kh, kw = w.shape
    m = jnp.transpose(w, (2, 3, 1, 0)).reshape(kh * kw * Cin, Cout)
    if bn:
        m = m * _BN_SCALE
    return m.astype(jnp.bfloat16)


def _bias_of(b, Cout, bn):
    bias = b if b is not None else jnp.zeros((Cout,), jnp.float32)
    bias = bias.astype(jnp.float32)
    return bias * _BN_SCALE if bn else bias


def _conv(x, w, b, stride=(1, 1), padding=((0, 0), (0, 0)), bn=False,
          relu=False, residual=None):
    N, H, W, Cin = x.shape
    Cout, _, kh, kw = w.shape
    sh, sw = stride
    (pt, pb), (plf, prt) = padding
    bias = _bias_of(b, Cout, bn)
    wmat = _wmat2d(w, bn)

    if kh == 1 and kw == 1 and sh == 1:
        M = N * H * W
        a = x.reshape(M, Cin)
        res2 = residual.reshape(M, Cout) if residual is not None else None
        out = _gemm([a], [wmat], bias, relu=relu, residual=res2)
        return out.reshape(N, H, W, Cout)
    ns = Cout if (Cout < 128 and Cout % 8 == 0) else None
    return _conv_implicit(x, wmat, bias, kh=kh, kw=kw, stride=sh, pad=pt,
                          relu=relu, residual=residual, n_store=ns)


def _conv1_7x7_s2(x, w):
    N, H, W, C = x.shape
    Cout = w.shape[0]
    xp = jnp.pad(x, ((0, 0), (3, 3), (3, 3), (0, 0)))
    Hs, Ws = (H + 6) // 2, (W + 6) // 2
    s2d = xp.reshape(N, Hs, 2, Ws, 2, C).transpose(0, 1, 3, 2, 4, 5)
    s2d = s2d.reshape(N, Hs, Ws, 4 * C)
    w8 = jnp.zeros((Cout, C, 8, 8), w.dtype).at[:, :, :7, :7].set(w)
    wr = w8.reshape(Cout, C, 4, 2, 4, 2).transpose(2, 4, 3, 5, 1, 0)
    wmat = (wr.reshape(16 * 4 * C, Cout) * _BN_SCALE).astype(jnp.bfloat16)
    bias = jnp.zeros((Cout,), jnp.float32)
    ns = Cout if (Cout < 128 and Cout % 8 == 0) else None
    return _conv_implicit(s2d, wmat, bias, kh=4, kw=4, stride=1, pad=0,
                          relu=True, n_store=ns)


def _bottleneck(x, bp, stride):
    if 'down' in bp:
        identity = _conv(x, bp['down'][0], None, (stride, stride),
                         bn=True)
    else:
        identity = x
    y = _conv(x, bp['conv1'][0], None, bn=True, relu=True)
    y = _conv(y, bp['conv2'][0], None, (stride, stride), ((1, 1), (1, 1)),
              bn=True, relu=True)
    return _conv(y, bp['conv3'][0], None, bn=True, relu=True,
                 residual=identity)


def _embed3(w):
    Cout, Cin, kh, kw = w.shape
    out = jnp.zeros((Cout, Cin, 3, 3), w.dtype)
    return out.at[:, :, 3 - kh:, 3 - kw:].set(w)


def _upproj_body(x_ref, *refs, bh, Wo, C):
    ws, bs = refs[0:8], refs[8:16]
    main_ref, res_ref = refs[16], refs[17]
    h0 = pl.program_id(2) * bh
    parts = []
    for i in range(3):
        rows = x_ref[0, pl.ds(h0 + i, bh), :, :]
        for j in range(3):
            parts.append(rows[:, j:j + Wo, :].reshape(bh * Wo, C))
    a = jnp.concatenate(parts, axis=-1)
    ys = []
    for g in range(8):
        acc = None
        for s, e in _kchunks(9 * C):
            d = jnp.dot(a[:, s:e], ws[g][s:e, :],
                        preferred_element_type=jnp.float32)
            acc = d if acc is None else acc + d
        ys.append(acc + bs[g][...])
    tc = ys[0].shape[-1]

    def weave(ts, relu):
        if relu:
            ts = [jnp.maximum(t, 0.0) for t in ts]
        A, B, Cc, D = [t.astype(main_ref.dtype).reshape(bh, Wo, tc)
                       for t in ts]
        r0 = jnp.concatenate([A[:, :, None, :], Cc[:, :, None, :]],
                             axis=2).reshape(bh, 2 * Wo, tc)
        r1 = jnp.concatenate([B[:, :, None, :], D[:, :, None, :]],
                             axis=2).reshape(bh, 2 * Wo, tc)
        return jnp.concatenate([r0[:, None], r1[:, None]],
                               axis=1).reshape(1, 2 * bh, 2 * Wo, tc)

    main_ref[...] = weave(ys[:4], True)
    res_ref[...] = weave(ys[4:], False)


def _up_projection(x, p):
    planes = p['conv'][0].shape[0]
    N, H, W, C = x.shape
    xp = jnp.pad(x, ((0, 0), (1, 1), (1, 1), (0, 0)))
    ws, bs = [], []
    for group in ('unpool_main', 'unpool_res'):
        for key in ('A', 'B', 'C', 'D'):
            w, bb = p[group][key]
            wm = jnp.transpose(_embed3(w), (2, 3, 1, 0)).reshape(9 * C, planes)
            ws.append((wm * _BN_SCALE).astype(jnp.bfloat16))
            bs.append(bb.astype(jnp.float32).reshape(1, planes) * _BN_SCALE)
    tc = min(128, planes)
    bh = _pick_bh(H, W)
    grid = (planes // tc, N, H // bh)
    in_specs = [pl.BlockSpec((1, H + 2, W + 2, C), lambda t, n, h: (n, 0, 0, 0))]
    in_specs += [pl.BlockSpec((9 * C, tc), lambda t, n, h: (0, t))] * 8
    in_specs += [pl.BlockSpec((1, tc), lambda t, n, h: (0, t))] * 8
    out_sds = jax.ShapeDtypeStruct((N, 2 * H, 2 * W, planes), jnp.bfloat16)
    out_spec = pl.BlockSpec((1, 2 * bh, 2 * W, tc), lambda t, n, h: (n, h, 0, t))
    main, res = pl.pallas_call(
        functools.partial(_upproj_body, bh=bh, Wo=W, C=C),
        out_shape=[out_sds, out_sds],
        grid=grid,
        in_specs=in_specs,
        out_specs=[out_spec, out_spec],
        compiler_params=_CPARAMS(("parallel", "parallel", "parallel")),
    )(xp, *ws, *bs)
    return _conv(main, p['conv'][0], p['conv'][1], (1, 1), ((1, 1), (1, 1)),
                 bn=True, relu=True, residual=res)


def _con_conv(x_dec, x_skip, p):
    w, b = p
    N, H, W, Cd = x_dec.shape
    Cs = x_skip.shape[-1]
    Cout = w.shape[0]
    wmat = _wmat2d(w, bn=False)
    out = _gemm([x_skip.reshape(N * H * W, Cs), x_dec.reshape(N * H * W, Cd)],
                [wmat[:Cs], wmat[Cs:]], b.astype(jnp.float32))
    return out.reshape(N, H, W, Cout)


def _forward(P, x_nchw):
    x = jnp.transpose(x_nchw, (0, 2, 3, 1)).astype(jnp.bfloat16)
    x = _conv1_7x7_s2(x, P['conv1'][0])
    x_to4 = x
    x = _maxpool(x)
    x_to3 = x
    for bi, bp in enumerate(P['layer1']):
        x_to3 = _bottleneck(x_to3, bp, 1)
    x_to2 = x_to3
    for bi, bp in enumerate(P['layer2']):
        x_to2 = _bottleneck(x_to2, bp, 2 if bi == 0 else 1)
    x_to1 = x_to2
    for bi, bp in enumerate(P['layer3']):
        x_to1 = _bottleneck(x_to1, bp, 2 if bi == 0 else 1)
    x = x_to1
    for bi, bp in enumerate(P['layer4']):
        x = _bottleneck(x, bp, 2 if bi == 0 else 1)
    x = _conv(x, P['conv2'][0], P['conv2'][1], bn=True)
    x = _up_projection(x, P['up_proj1'])
    x = _con_conv(x, x_to1, P['con_conv1'])
    x = _up_projection(x, P['up_proj2'])
    x = _con_conv(x, x_to2, P['con_conv2'])
    x = _up_projection(x, P['up_proj3'])
    x = _con_conv(x, x_to3, P['con_conv3'])
    x = _up_projection(x, P['up_proj4'])
    x = _con_conv(x, x_to4, P['con_conv4'])
    wmat = _wmat2d(P['conv3'][0], bn=False)
    out = _conv_implicit(x, wmat, P['conv3'][1].astype(jnp.float32),
                         relu=True, n_store=8, out_dtype=jnp.bfloat16)
    out = out[..., :1]
    return jnp.transpose(out, (0, 3, 1, 2)).astype(jnp.float32)


_BLOCKS = (('layer1', 3), ('layer2', 4), ('layer3', 6), ('layer4', 3))


def _unpack(leaves):
    it = iter(leaves)
    nxt = lambda: next(it)
    P = {'conv1': (nxt(), None)}
    for lname, nb in _BLOCKS:
        blocks = []
        for bi in range(nb):
            bp = {'conv1': (nxt(), None), 'conv2': (nxt(), None),
                  'conv3': (nxt(), None)}
            if bi == 0:
                bp['down'] = (nxt(), None)
            blocks.append(bp)
        P[lname] = blocks
    P['conv2'] = (nxt(), nxt())
    for up in ('up_proj1', 'up_proj2', 'up_proj3', 'up_proj4'):
        p = {'unpool_main': {}, 'unpool_res': {}}
        for g in ('unpool_main', 'unpool_res'):
            for k in ('A', 'B', 'C', 'D'):
                p[g][k] = (nxt(), nxt())
        p['conv'] = (nxt(), nxt())
        P[up] = p
    P['conv3'] = (nxt(), nxt())
    for c in ('con_conv1', 'con_conv2', 'con_conv3', 'con_conv4'):
        P[c] = (nxt(), nxt())
    return P


def kernel(*args):
    leaves = list(args[:137])
    x = args[137]
    return _forward(_unpack(leaves), x)
